# Initial kernel scaffold; baseline (speedup 1.0000x reference)
import functools

import jax
import jax.numpy as jnp
from jax import lax
from jax.experimental import pallas as pl
from jax.experimental.pallas import tpu as pltpu
from jax.experimental.pallas import tpu_sc as plsc

NC = 2
NS = 16
NW = NC * NS
L = 16

EPS = 1e-12


def _rsqrt(x):
    i = lax.bitcast_convert_type(x, jnp.int32)
    i = jnp.int32(0x5F3759DF) - (i >> 1)
    y = lax.bitcast_convert_type(i, jnp.float32)
    for _ in range(3):
        y = y * (1.5 - 0.5 * x * y * y)
    return y


def _make_kernel(B, S, E, rows_per_w):
    EB = E // L

    def body(ids_hbm, tt_hbm, word_hbm, pos_hbm, ttemb_hbm, gamma_hbm,
             beta_hbm, out_hbm, pos_v, ttemb_v, gam_v, bet_v, ids_v, ttid_v,
             rows_v, gsem):
        wid = lax.axis_index("s") * NC + lax.axis_index("c")
        base_row = wid * rows_per_w

        pltpu.sync_copy(pos_hbm.at[pl.ds(0, S)], pos_v)
        pltpu.sync_copy(ttemb_hbm, ttemb_v)
        pltpu.sync_copy(gamma_hbm, gam_v)
        pltpu.sync_copy(beta_hbm, bet_v)

        tt0 = [ttemb_v[0, pl.ds(e * L, L)] for e in range(EB)]
        tt1 = [ttemb_v[1, pl.ds(e * L, L)] for e in range(EB)]
        gam = [gam_v[pl.ds(e * L, L)] for e in range(EB)]
        bet = [bet_v[pl.ds(e * L, L)] for e in range(EB)]

        def chunk_body(i, carry):
            row = base_row + i
            pltpu.sync_copy(ids_hbm.at[row], ids_v)
            pltpu.sync_copy(tt_hbm.at[row], ttid_v)
            cp0 = pltpu.async_copy(
                word_hbm.at[ids_v.at[pl.ds(0, 128)]],
                rows_v.at[pl.ds(0, 128)], gsem)
            cp1 = pltpu.async_copy(
                word_hbm.at[ids_v.at[pl.ds(128, S - 128)]],
                rows_v.at[pl.ds(128, S - 128)], gsem)
            cp0.wait()
            cp1.wait()

            def tok_body(j, c):
                ttsel = ttid_v[j] == 1
                v = []
                for e in range(EB):
                    x = rows_v[j, pl.ds(e * L, L)]
                    p = pos_v[j, pl.ds(e * L, L)]
                    t = jnp.where(ttsel, tt1[e], tt0[e])
                    v.append(x + p + t)
                sv = v[0] + v[1]
                for e in range(2, EB):
                    sv = sv + v[e]
                qv = v[0] * v[0]
                for e in range(1, EB):
                    qv = qv + v[e] * v[e]
                s1 = jnp.broadcast_to(jnp.sum(sv), (L,))
                s2 = jnp.broadcast_to(jnp.sum(qv), (L,))
                mean = s1 * (1.0 / E)
                var = s2 * (1.0 / E) - mean * mean
                r = _rsqrt(var + EPS)
                for e in range(EB):
                    rg = r * gam[e]
                    cst = bet[e] - mean * rg
                    rows_v[j, pl.ds(e * L, L)] = v[e] * rg + cst
                return c

            lax.fori_loop(0, S, tok_body, 0)
            pltpu.sync_copy(rows_v, out_hbm.at[row])
            return carry

        lax.fori_loop(0, rows_per_w, chunk_body, 0)

    mesh = plsc.VectorSubcoreMesh(core_axis_name="c", subcore_axis_name="s")
    return pl.kernel(
        body,
        out_type=jax.ShapeDtypeStruct((B, S, E), jnp.float32),
        mesh=mesh,
        scratch_types=[
            pltpu.VMEM((S, E), jnp.float32),
            pltpu.VMEM((2, E), jnp.float32),
            pltpu.VMEM((E,), jnp.float32),
            pltpu.VMEM((E,), jnp.float32),
            pltpu.VMEM((S,), jnp.int32),
            pltpu.VMEM((S,), jnp.int32),
            pltpu.VMEM((S, E), jnp.float32),
            pltpu.SemaphoreType.DMA,
        ],
    )


@jax.jit
def kernel(input_ids, token_type_ids, word_embeddings, position_embeddings,
           token_type_embeddings, gamma, beta):
    B, S = input_ids.shape
    E = word_embeddings.shape[1]
    rows_per_w = B // NW
    k = _make_kernel(B, S, E, rows_per_w)
    return k(input_ids.astype(jnp.int32), token_type_ids.astype(jnp.int32),
             word_embeddings, position_embeddings, token_type_embeddings,
             gamma, beta)


# speedup vs baseline: 3.5337x; 3.5337x over previous
import jax
import jax.numpy as jnp
from jax import lax
from jax.experimental import pallas as pl
from jax.experimental.pallas import tpu as pltpu
from jax.experimental.pallas import tpu_sc as plsc

NC = 2
NS = 16
NW = NC * NS
L = 16

EPS = 1e-12


def _rsqrt(x):
    i = lax.bitcast_convert_type(x, jnp.int32)
    i = jnp.int32(0x5F3759DF) - (i >> 1)
    y = lax.bitcast_convert_type(i, jnp.float32)
    for _ in range(3):
        y = y * (1.5 - 0.5 * x * y * y)
    return y


def _make_kernel(B, S, E, rows_per_w):
    EB = E // L
    SP = ((S + L - 1) // L) * L
    NBLK = SP // L

    def body(ids_hbm, tt_hbm, word_hbm, pos_hbm, ttemb_hbm, gamma_hbm,
             beta_hbm, out_hbm, pos_v, ttemb_v, gam_v, bet_v, ids_v, ttid_v,
             rows_v, gsem):
        wid = lax.axis_index("s") * NC + lax.axis_index("c")
        base_row = wid * rows_per_w

        pltpu.sync_copy(pos_hbm.at[pl.ds(0, S)], pos_v.at[pl.ds(0, S)])
        pltpu.sync_copy(ttemb_hbm, ttemb_v)
        pltpu.sync_copy(gamma_hbm, gam_v)
        pltpu.sync_copy(beta_hbm, bet_v)

        tt0 = [ttemb_v[0, pl.ds(e * L, L)] for e in range(EB)]
        tt1 = [ttemb_v[1, pl.ds(e * L, L)] for e in range(EB)]
        gam = [gam_v[pl.ds(e * L, L)] for e in range(EB)]
        bet = [bet_v[pl.ds(e * L, L)] for e in range(EB)]

        def chunk_body(i, carry):
            row = base_row + i
            pltpu.sync_copy(ids_hbm.at[pl.ds(row * S, S)], ids_v)
            pltpu.sync_copy(tt_hbm.at[pl.ds(row * S, S)], ttid_v.at[pl.ds(0, S)])
            cp0 = pltpu.async_copy(
                word_hbm.at[ids_v.at[pl.ds(0, 128)]],
                rows_v.at[pl.ds(0, 128)], gsem)
            cp1 = pltpu.async_copy(
                word_hbm.at[ids_v.at[pl.ds(128, S - 128)]],
                rows_v.at[pl.ds(128, S - 128)], gsem)
            cp0.wait()
            cp1.wait()

            def blk_body(b, c):
                tv = ttid_v[pl.ds(b * L, L)]
                for k in range(L):
                    j = b * L + k
                    ttsel = tv[k] == 1
                    v = []
                    for e in range(EB):
                        x = rows_v[j, pl.ds(e * L, L)]
                        p = pos_v[j, pl.ds(e * L, L)]
                        t = jnp.where(ttsel, tt1[e], tt0[e])
                        v.append(x + p + t)
                    sv = v[0] + v[1]
                    for e in range(2, EB):
                        sv = sv + v[e]
                    qv = v[0] * v[0]
                    for e in range(1, EB):
                        qv = qv + v[e] * v[e]
                    s1 = jnp.broadcast_to(jnp.sum(sv), (L,))
                    s2 = jnp.broadcast_to(jnp.sum(qv), (L,))
                    mean = s1 * (1.0 / E)
                    var = s2 * (1.0 / E) - mean * mean
                    r = _rsqrt(var + EPS)
                    for e in range(EB):
                        rg = r * gam[e]
                        cst = bet[e] - mean * rg
                        rows_v[j, pl.ds(e * L, L)] = v[e] * rg + cst
                return c

            lax.fori_loop(0, NBLK, blk_body, 0)
            pltpu.sync_copy(rows_v.at[pl.ds(0, S)],
                            out_hbm.at[pl.ds(row * S, S)])
            return carry

        lax.fori_loop(0, rows_per_w, chunk_body, 0)

    mesh = plsc.VectorSubcoreMesh(core_axis_name="c", subcore_axis_name="s")
    return pl.kernel(
        body,
        out_type=jax.ShapeDtypeStruct((B * S, E), jnp.float32),
        mesh=mesh,
        compiler_params=pltpu.CompilerParams(needs_layout_passes=False),
        scratch_types=[
            pltpu.VMEM((SP, E), jnp.float32),
            pltpu.VMEM((2, E), jnp.float32),
            pltpu.VMEM((E,), jnp.float32),
            pltpu.VMEM((E,), jnp.float32),
            pltpu.VMEM((S,), jnp.int32),
            pltpu.VMEM((SP,), jnp.int32),
            pltpu.VMEM((SP, E), jnp.float32),
            pltpu.SemaphoreType.DMA,
        ],
    )


@jax.jit
def kernel(input_ids, token_type_ids, word_embeddings, position_embeddings,
           token_type_embeddings, gamma, beta):
    B, S = input_ids.shape
    E = word_embeddings.shape[1]
    rows_per_w = B // NW
    k = _make_kernel(B, S, E, rows_per_w)
    out = k(input_ids.astype(jnp.int32).reshape(-1),
            token_type_ids.astype(jnp.int32).reshape(-1),
            word_embeddings, position_embeddings, token_type_embeddings,
            gamma, beta)
    return out.reshape(B, S, E)


# speedup vs baseline: 3.8065x; 1.0772x over previous
import jax
import jax.numpy as jnp
from jax import lax
from jax.experimental import pallas as pl
from jax.experimental.pallas import tpu as pltpu
from jax.experimental.pallas import tpu_sc as plsc

NC = 2
NS = 16
NW = NC * NS
L = 16

EPS = 1e-12


def _rsqrt(x):
    i = lax.bitcast_convert_type(x, jnp.int32)
    i = jnp.int32(0x5F3759DF) - (i >> 1)
    y = lax.bitcast_convert_type(i, jnp.float32)
    for _ in range(3):
        y = y * (1.5 - 0.5 * x * y * y)
    return y


def _make_kernel(B, S, E, rows_per_w):
    EB = E // L
    SP = ((S + L - 1) // L) * L
    NBLK = SP // L
    N = rows_per_w
    NB3 = (N + 2) // 3

    def body(ids_hbm, tt_hbm, word_hbm, pos_hbm, ttemb_hbm, gamma_hbm,
             beta_hbm, out_hbm,
             pos_v, ttemb_v, gam_v, bet_v,
             ids0, ids1, ids2, ttid0, ttid1, ttid2,
             rows0, rows1, rows2,
             gsem0, gsem1, gsem2, osem0, osem1, osem2):
        wid = lax.axis_index("s") * NC + lax.axis_index("c")
        base_row = wid * N

        slots = [
            (ids0, ttid0, rows0, gsem0, osem0),
            (ids1, ttid1, rows1, gsem1, osem1),
            (ids2, ttid2, rows2, gsem2, osem2),
        ]

        def ids_load(c, sl):
            ids_v, ttid_v = sl[0], sl[1]
            row = base_row + c
            pltpu.sync_copy(ids_hbm.at[pl.ds(row * S, S)], ids_v)
            pltpu.sync_copy(tt_hbm.at[pl.ds(row * S, S)],
                            ttid_v.at[pl.ds(0, S)])

        def gather_copies(sl):
            ids_v, rows_v, gsem = sl[0], sl[2], sl[3]
            c0 = pltpu.make_async_copy(
                word_hbm.at[ids_v.at[pl.ds(0, 128)]],
                rows_v.at[pl.ds(0, 128)], gsem)
            c1 = pltpu.make_async_copy(
                word_hbm.at[ids_v.at[pl.ds(128, S - 128)]],
                rows_v.at[pl.ds(128, S - 128)], gsem)
            return c0, c1

        def gather_start(sl):
            for cp in gather_copies(sl):
                cp.start()

        def gather_wait(sl):
            for cp in gather_copies(sl):
                cp.wait()

        def out_copy(c, sl):
            rows_v, osem = sl[2], sl[4]
            row = base_row + c
            return pltpu.make_async_copy(
                rows_v.at[pl.ds(0, S)],
                out_hbm.at[pl.ds(row * S, S)], osem)

        pltpu.sync_copy(pos_hbm.at[pl.ds(0, S)], pos_v.at[pl.ds(0, S)])
        pltpu.sync_copy(ttemb_hbm, ttemb_v)
        pltpu.sync_copy(gamma_hbm, gam_v)
        pltpu.sync_copy(beta_hbm, bet_v)

        tt0 = [ttemb_v[0, pl.ds(e * L, L)] for e in range(EB)]
        tt1 = [ttemb_v[1, pl.ds(e * L, L)] for e in range(EB)]
        gam = [gam_v[pl.ds(e * L, L)] for e in range(EB)]
        bet = [bet_v[pl.ds(e * L, L)] for e in range(EB)]

        def compute(sl):
            ttid_v, rows_v = sl[1], sl[2]

            def blk_body(b, cc):
                tv = ttid_v[pl.ds(b * L, L)]
                for k in range(L):
                    j = b * L + k
                    ttsel = tv[k] == 1
                    v = []
                    for e in range(EB):
                        x = rows_v[j, pl.ds(e * L, L)]
                        p = pos_v[j, pl.ds(e * L, L)]
                        t = jnp.where(ttsel, tt1[e], tt0[e])
                        v.append(x + p + t)
                    sv = v[0] + v[1]
                    for e in range(2, EB):
                        sv = sv + v[e]
                    qv = v[0] * v[0]
                    for e in range(1, EB):
                        qv = qv + v[e] * v[e]
                    s1 = jnp.broadcast_to(jnp.sum(sv), (L,))
                    s2 = jnp.broadcast_to(jnp.sum(qv), (L,))
                    mean = s1 * (1.0 / E)
                    var = s2 * (1.0 / E) - mean * mean
                    r = _rsqrt(var + EPS)
                    for e in range(EB):
                        rg = r * gam[e]
                        cst = bet[e] - mean * rg
                        rows_v[j, pl.ds(e * L, L)] = v[e] * rg + cst
                return cc

            lax.fori_loop(0, NBLK, blk_body, 0)

        ids_load(0, slots[0])
        ids_load(1, slots[1])
        ids_load(2, slots[2])
        gather_start(slots[0])
        gather_start(slots[1])

        def loop_body(p, carry):
            cb = p * 3
            for k in range(3):
                c = cb + k
                sl = slots[k]
                sl2 = slots[(k + 2) % 3]

                @pl.when(c < N)
                def _():
                    gather_wait(sl)
                    compute(sl)
                    out_copy(c, sl).start()

                @pl.when(c + 3 < N)
                def _():
                    ids_load(c + 3, sl)

                @pl.when((c >= 1) & (c < N))
                def _():
                    out_copy(c - 1, sl2).wait()

                @pl.when(c + 2 < N)
                def _():
                    gather_start(sl2)
            return carry

        lax.fori_loop(0, NB3, loop_body, 0)
        out_copy(N - 1, slots[(N - 1) % 3]).wait()

    mesh = plsc.VectorSubcoreMesh(core_axis_name="c", subcore_axis_name="s")
    return pl.kernel(
        body,
        out_type=jax.ShapeDtypeStruct((B * S, E), jnp.float32),
        mesh=mesh,
        compiler_params=pltpu.CompilerParams(needs_layout_passes=False),
        scratch_types=[
            pltpu.VMEM((SP, E), jnp.float32),
            pltpu.VMEM((2, E), jnp.float32),
            pltpu.VMEM((E,), jnp.float32),
            pltpu.VMEM((E,), jnp.float32),
            pltpu.VMEM((S,), jnp.int32),
            pltpu.VMEM((S,), jnp.int32),
            pltpu.VMEM((S,), jnp.int32),
            pltpu.VMEM((SP,), jnp.int32),
            pltpu.VMEM((SP,), jnp.int32),
            pltpu.VMEM((SP,), jnp.int32),
            pltpu.VMEM((SP, E), jnp.float32),
            pltpu.VMEM((SP, E), jnp.float32),
            pltpu.VMEM((SP, E), jnp.float32),
            pltpu.SemaphoreType.DMA,
            pltpu.SemaphoreType.DMA,
            pltpu.SemaphoreType.DMA,
            pltpu.SemaphoreType.DMA,
            pltpu.SemaphoreType.DMA,
            pltpu.SemaphoreType.DMA,
        ],
    )


@jax.jit
def kernel(input_ids, token_type_ids, word_embeddings, position_embeddings,
           token_type_embeddings, gamma, beta):
    B, S = input_ids.shape
    E = word_embeddings.shape[1]
    rows_per_w = B // NW
    k = _make_kernel(B, S, E, rows_per_w)
    out = k(input_ids.astype(jnp.int32).reshape(-1),
            token_type_ids.astype(jnp.int32).reshape(-1),
            word_embeddings, position_embeddings, token_type_embeddings,
            gamma, beta)
    return out.reshape(B, S, E)
